# baseline (speedup 1.0000x reference)
import functools

import jax
import jax.numpy as jnp
from jax import lax
from jax.experimental import pallas as pl
from jax.experimental.pallas import tpu as pltpu
from jax.experimental.pallas import tpu_sc as plsc

N = 10000
E = 320000
D_IN = 128
H = 64
H2 = 2 * H
N2 = N // 2

NC = 2
NS = 16
NW = NC * NS
CB = 128
NCHUNK = E // CB
BASE_CH = NCHUNK // NW
EXTRA_W = NCHUNK - BASE_CH * NW
RPT = N // NS
DEGW = 8
ND = 10240
RPTD = ND // NS
BR = 2000
BR2 = BR // 2

_MESH = plsc.VectorSubcoreMesh(
    core_axis_name="c", subcore_axis_name="s",
    num_cores=NC, num_subcores=NS)
_SC_PARAMS = pltpu.CompilerParams(use_tc_tiling_on_sc=False)


def _worker_chunks(wid):
    start = BASE_CH * wid + jnp.maximum(wid - (NW - EXTRA_W), 0)
    has_extra = wid >= NW - EXTRA_W
    return start, has_extra


def _deg_body(e_hbm, z8_hbm, ones_hbm, degp_hbm, dst_v, ones_v, deg_sh,
              gsem):
    cid = lax.axis_index("c")
    sid = lax.axis_index("s")
    wid = sid * NC + cid
    based = sid * RPTD
    start, has_extra = _worker_chunks(wid)

    pltpu.sync_copy(z8_hbm, deg_sh.at[pl.ds(based, RPTD)])
    pltpu.sync_copy(ones_hbm, ones_v)
    pltpu.sync_copy(e_hbm.at[1, pl.ds(start, BASE_CH + 1)], dst_v)
    plsc.subcore_barrier()

    def _scat(j):
        pltpu.async_copy(ones_v, deg_sh.at[dst_v.at[j]], gsem, add=True)

    def _swait():
        pltpu.make_async_copy(ones_v, deg_sh.at[dst_v.at[0]], gsem).wait()

    _scat(0)

    def _step(j, _):
        _scat(j + 1)
        _swait()
        return 0
    lax.fori_loop(0, BASE_CH - 1, _step, 0)
    _swait()

    @pl.when(has_extra)
    def _tail():
        pltpu.async_copy(ones_v, deg_sh.at[dst_v.at[BASE_CH]], gsem,
                         add=True).wait()

    plsc.subcore_barrier()
    pltpu.sync_copy(deg_sh.at[pl.ds(based, RPTD)],
                    degp_hbm.at[cid, pl.ds(based, RPTD)])


_deg_seg = pl.kernel(
    _deg_body,
    out_type=jax.ShapeDtypeStruct((NC, ND, DEGW), jnp.float32),
    mesh=_MESH,
    scratch_types=[
        pltpu.VMEM((BASE_CH + 1, CB), jnp.int32),
        pltpu.VMEM((CB, DEGW), jnp.float32),
        pltpu.VMEM_SHARED((ND, DEGW), jnp.float32),
        pltpu.SemaphoreType.DMA,
    ],
    compiler_params=_SC_PARAMS)


def _make_seg(echo_h):
    agg_ty = jax.ShapeDtypeStruct((NC, N, H), jnp.float32)
    if echo_h:
        out_type = (agg_ty, jax.ShapeDtypeStruct((N, H), jnp.float32))
    else:
        out_type = agg_ty
    scratch = [
        pltpu.VMEM((BASE_CH + 1, CB), jnp.int32),
        pltpu.VMEM((BASE_CH + 1, CB), jnp.int32),
        pltpu.VMEM((CB, H), jnp.float32),
        pltpu.VMEM((CB, H), jnp.float32),
        pltpu.VMEM((CB, H), jnp.float32),
        pltpu.VMEM_SHARED((N, H), jnp.float32),
        pltpu.VMEM_SHARED((N, H), jnp.float32),
        pltpu.SemaphoreType.DMA,
        pltpu.SemaphoreType.DMA,
        pltpu.SemaphoreType.DMA,
    ]

    def body(*refs):
        if echo_h:
            (h_hbm, e_hbm, z64_hbm, aggp_hbm, echo_hbm,
             src_v, dst_v, rows0, rows1, rows2, agg_sh, h_sh,
             sem0, sem1, sem2) = refs
        else:
            (h_hbm, e_hbm, z64_hbm, aggp_hbm,
             src_v, dst_v, rows0, rows1, rows2, agg_sh, h_sh,
             sem0, sem1, sem2) = refs
            echo_hbm = None

        cid = lax.axis_index("c")
        sid = lax.axis_index("s")
        wid = sid * NC + cid
        base = sid * RPT
        start, has_extra = _worker_chunks(wid)

        pltpu.sync_copy(z64_hbm, agg_sh.at[pl.ds(base, RPT)])
        pltpu.sync_copy(e_hbm.at[0, pl.ds(start, BASE_CH + 1)], src_v)
        pltpu.sync_copy(e_hbm.at[1, pl.ds(start, BASE_CH + 1)], dst_v)
        pltpu.sync_copy(h_hbm.at[pl.ds(base, RPT)], h_sh.at[pl.ds(base, RPT)])

        plsc.subcore_barrier()

        def _gather(j, buf, sem, src=None):
            src = h_sh if src is None else src
            pltpu.async_copy(src.at[src_v.at[j]], buf, sem)

        def _gwait(buf, sem):
            pltpu.make_async_copy(h_sh.at[src_v.at[0]], buf, sem).wait()

        def _scatter(j, buf):
            pltpu.sync_copy(buf, agg_sh.at[dst_v.at[j]], add=True)

        _gather(0, rows0, sem0)
        _gather(1, rows1, sem1)

        def _chunk3(t, _):
            j0 = 3 * t
            _gwait(rows0, sem0)
            _gather(j0 + 2, rows2, sem2, src=h_hbm)
            _scatter(j0, rows0)
            _gwait(rows1, sem1)
            _gather(jnp.minimum(j0 + 3, BASE_CH - 1), rows0, sem0)
            _scatter(j0 + 1, rows1)
            _gwait(rows2, sem2)
            _gather(jnp.minimum(j0 + 4, BASE_CH - 1), rows1, sem1)
            _scatter(j0 + 2, rows2)
            return 0
        lax.fori_loop(0, BASE_CH // 3, _chunk3, 0)
        _gwait(rows0, sem0)
        _gwait(rows1, sem1)

        @pl.when(has_extra)
        def _tail():
            pltpu.async_copy(h_sh.at[src_v.at[BASE_CH]], rows0,
                             sem0).wait()
            _scatter(BASE_CH, rows0)

        plsc.subcore_barrier()

        pltpu.sync_copy(agg_sh.at[pl.ds(base, RPT)],
                        aggp_hbm.at[cid, pl.ds(base, RPT)])
        if echo_h:
            @pl.when(cid == 0)
            def _echo():
                pltpu.sync_copy(h_sh.at[pl.ds(base, RPT)],
                                echo_hbm.at[pl.ds(base, RPT)])

    return pl.kernel(body, out_type=out_type, mesh=_MESH,
                     scratch_types=scratch, compiler_params=_SC_PARAMS)


_seg_echo = _make_seg(True)
_seg_plain = _make_seg(False)


def _proj_body(x_ref, w_ref, b_ref, o_ref):
    o_ref[...] = jnp.maximum(
        jnp.dot(x_ref[...], w_ref[...], preferred_element_type=jnp.float32)
        + b_ref[...], 0.0)


def _inv_body(degp_ref, o_ref):
    o_ref[...] = jnp.maximum(degp_ref[0] + degp_ref[1], 1.0)


def _pair_mean(aggp_ref, inv_ref):
    a = aggp_ref[0] + aggp_ref[1]
    iv = inv_ref[...]
    degb = jnp.concatenate(
        [jnp.broadcast_to(iv[:, 0:1], (a.shape[0], H)),
         jnp.broadcast_to(iv[:, DEGW:DEGW + 1], (a.shape[0], H))], axis=1)
    return a / degb


def _upd_body(aggp_ref, inv_ref, h_ref, wl_ref, bl_ref, wr_ref, o_ref):
    mean2 = _pair_mean(aggp_ref, inv_ref)
    o_ref[...] = jnp.maximum(
        jnp.dot(mean2, wl_ref[...], preferred_element_type=jnp.float32)
        + jnp.dot(h_ref[...], wr_ref[...], preferred_element_type=jnp.float32)
        + bl_ref[...], 0.0)


def _head_body(aggp_ref, inv_ref, h_ref, wl_ref, bl_ref, wr_ref,
               wc_ref, bc_ref, o_ref):
    mean2 = _pair_mean(aggp_ref, inv_ref)
    h2 = jnp.maximum(
        jnp.dot(mean2, wl_ref[...], preferred_element_type=jnp.float32)
        + jnp.dot(h_ref[...], wr_ref[...], preferred_element_type=jnp.float32)
        + bl_ref[...], 0.0)
    s = h2 * wc_ref[...]
    o_ref[...] = jnp.concatenate(
        [jnp.sum(s[:, :H], axis=1, keepdims=True),
         jnp.sum(s[:, H:], axis=1, keepdims=True)], axis=1) + bc_ref[0, 0]


def _full(shape):
    return pl.BlockSpec(shape, lambda i: tuple(0 for _ in shape))


def _blockdiag(w):
    z = jnp.zeros((H, H), jnp.float32)
    return jnp.concatenate(
        [jnp.concatenate([w, z], axis=1),
         jnp.concatenate([z, w], axis=1)], axis=0)


def kernel(x, edge_index, W_proj, b_proj, Wl1, bl1, Wr1, Wl2, bl2, Wr2,
           Wc, bc):
    e3 = edge_index.reshape(2, NCHUNK, CB)
    z64 = jnp.zeros((RPT, H), jnp.float32)
    z8 = jnp.zeros((RPTD, DEGW), jnp.float32)
    ones = jnp.ones((CB, DEGW), jnp.float32)

    degp = _deg_seg(e3, z8, ones)

    grid = (N // BR,)
    h = pl.pallas_call(
        _proj_body,
        grid=grid,
        in_specs=[pl.BlockSpec((BR, D_IN), lambda i: (i, 0)),
                  _full((D_IN, H)), _full((1, H))],
        out_specs=pl.BlockSpec((BR, H), lambda i: (i, 0)),
        out_shape=jax.ShapeDtypeStruct((N, H), jnp.float32),
    )(x, W_proj, b_proj.reshape(1, H))

    inv128 = pl.pallas_call(
        _inv_body,
        grid=(1,),
        in_specs=[_full((NC, ND * DEGW // H2, H2))],
        out_specs=pl.BlockSpec((ND * DEGW // H2, H2), lambda i: (0, 0)),
        out_shape=jax.ShapeDtypeStruct((ND * DEGW // H2, H2), jnp.float32),
    )(degp.reshape(NC, ND * DEGW // H2, H2))
    inv16 = inv128.reshape(ND // 2, 2 * DEGW)

    aggp1, h_echo = _seg_echo(h, e3, z64)

    aggp2_spec = pl.BlockSpec((NC, BR2, H2), lambda i: (0, i, 0))
    inv_spec = pl.BlockSpec((BR2, 2 * DEGW), lambda i: (i, 0))
    row2_spec = pl.BlockSpec((BR2, H2), lambda i: (i, 0))

    h1_2d = pl.pallas_call(
        _upd_body,
        grid=grid,
        in_specs=[aggp2_spec, inv_spec, row2_spec,
                  _full((H2, H2)), _full((1, H2)), _full((H2, H2))],
        out_specs=row2_spec,
        out_shape=jax.ShapeDtypeStruct((N2, H2), jnp.float32),
    )(aggp1.reshape(NC, N2, H2), inv16, h_echo.reshape(N2, H2),
      _blockdiag(Wl1), jnp.concatenate([bl1, bl1]).reshape(1, H2),
      _blockdiag(Wr1))

    aggp2 = _seg_plain(h1_2d.reshape(N, H), e3, z64)

    out2 = pl.pallas_call(
        _head_body,
        grid=grid,
        in_specs=[aggp2_spec, inv_spec, row2_spec,
                  _full((H2, H2)), _full((1, H2)), _full((H2, H2)),
                  _full((1, H2)), _full((1, 1))],
        out_specs=pl.BlockSpec((BR2, 2), lambda i: (i, 0)),
        out_shape=jax.ShapeDtypeStruct((N2, 2), jnp.float32),
    )(aggp2.reshape(NC, N2, H2), inv16, h1_2d,
      _blockdiag(Wl2), jnp.concatenate([bl2, bl2]).reshape(1, H2),
      _blockdiag(Wr2), jnp.concatenate([Wc[:, 0], Wc[:, 0]]).reshape(1, H2),
      bc.reshape(1, 1))

    return out2.reshape(N)


# speedup vs baseline: 1.0327x; 1.0327x over previous
import jax
import jax.numpy as jnp
from jax import lax
from jax.experimental import pallas as pl
from jax.experimental.pallas import tpu as pltpu
from jax.experimental.pallas import tpu_sc as plsc

N = 10000
E = 320000
D_IN = 128
H = 64
H2 = 2 * H
N2 = N // 2

NC = 2
NS = 16
NW = NC * NS
CB = 128
NCHUNK = E // CB
BASE_CH = NCHUNK // NW
EXTRA_W = NCHUNK - BASE_CH * NW
RPT = N // NS
DEGW = 8
ND = 10240
RPTD = ND // NS
BR = 2000
BR2 = BR // 2

_MESH = plsc.VectorSubcoreMesh(
    core_axis_name="c", subcore_axis_name="s",
    num_cores=NC, num_subcores=NS)
_SC_PARAMS = pltpu.CompilerParams(use_tc_tiling_on_sc=False)


def _worker_chunks(wid):
    start = BASE_CH * wid + jnp.maximum(wid - (NW - EXTRA_W), 0)
    has_extra = wid >= NW - EXTRA_W
    return start, has_extra


def _deg_body(e_hbm, z8_hbm, ones_hbm, degp_hbm, dst_v, ones_v, deg_sh,
              gsem):
    cid = lax.axis_index("c")
    sid = lax.axis_index("s")
    wid = sid * NC + cid
    based = sid * RPTD
    start, has_extra = _worker_chunks(wid)

    pltpu.sync_copy(z8_hbm, deg_sh.at[pl.ds(based, RPTD)])
    pltpu.sync_copy(ones_hbm, ones_v)
    pltpu.sync_copy(e_hbm.at[1, pl.ds(start, BASE_CH + 1)], dst_v)
    plsc.subcore_barrier()

    def _scat(j):
        pltpu.async_copy(ones_v, deg_sh.at[dst_v.at[j]], gsem, add=True)

    def _swait():
        pltpu.make_async_copy(ones_v, deg_sh.at[dst_v.at[0]], gsem).wait()

    _scat(0)

    def _step(j, _):
        _scat(j + 1)
        _swait()
        return 0
    lax.fori_loop(0, BASE_CH - 1, _step, 0)
    _swait()

    @pl.when(has_extra)
    def _tail():
        pltpu.async_copy(ones_v, deg_sh.at[dst_v.at[BASE_CH]], gsem,
                         add=True).wait()

    plsc.subcore_barrier()
    pltpu.sync_copy(deg_sh.at[pl.ds(based, RPTD)],
                    degp_hbm.at[cid, pl.ds(based, RPTD)])


_deg_seg = pl.kernel(
    _deg_body,
    out_type=jax.ShapeDtypeStruct((NC, ND, DEGW), jnp.float32),
    mesh=_MESH,
    scratch_types=[
        pltpu.VMEM((BASE_CH + 1, CB), jnp.int32),
        pltpu.VMEM((CB, DEGW), jnp.float32),
        pltpu.VMEM_SHARED((ND, DEGW), jnp.float32),
        pltpu.SemaphoreType.DMA,
    ],
    compiler_params=_SC_PARAMS)


def _make_seg(echo_h):
    agg_ty = jax.ShapeDtypeStruct((NC, N, H), jnp.float32)
    if echo_h:
        out_type = (agg_ty, jax.ShapeDtypeStruct((N, H), jnp.float32))
    else:
        out_type = agg_ty
    scratch = [
        pltpu.VMEM((BASE_CH + 1, CB), jnp.int32),
        pltpu.VMEM((BASE_CH + 1, CB), jnp.int32),
        pltpu.VMEM((CB, H), jnp.float32),
        pltpu.VMEM((CB, H), jnp.float32),
        pltpu.VMEM((CB, H), jnp.float32),
        pltpu.VMEM_SHARED((N, H), jnp.float32),
        pltpu.VMEM_SHARED((N, H), jnp.float32),
        pltpu.SemaphoreType.DMA,
        pltpu.SemaphoreType.DMA,
        pltpu.SemaphoreType.DMA,
    ]

    def body(*refs):
        if echo_h:
            (h_hbm, e_hbm, z64_hbm, aggp_hbm, echo_hbm,
             src_v, dst_v, rows0, rows1, rows2, agg_sh, h_sh,
             sem0, sem1, sem2) = refs
        else:
            (h_hbm, e_hbm, z64_hbm, aggp_hbm,
             src_v, dst_v, rows0, rows1, rows2, agg_sh, h_sh,
             sem0, sem1, sem2) = refs
            echo_hbm = None

        cid = lax.axis_index("c")
        sid = lax.axis_index("s")
        wid = sid * NC + cid
        base = sid * RPT
        start, has_extra = _worker_chunks(wid)

        pltpu.sync_copy(z64_hbm, agg_sh.at[pl.ds(base, RPT)])
        pltpu.sync_copy(e_hbm.at[0, pl.ds(start, BASE_CH + 1)], src_v)
        pltpu.sync_copy(e_hbm.at[1, pl.ds(start, BASE_CH + 1)], dst_v)
        pltpu.sync_copy(h_hbm.at[pl.ds(base, RPT)], h_sh.at[pl.ds(base, RPT)])

        plsc.subcore_barrier()

        def _gather(j, buf, sem):
            pltpu.async_copy(h_sh.at[src_v.at[j]], buf, sem)

        def _gwait(buf, sem):
            pltpu.make_async_copy(h_sh.at[src_v.at[0]], buf, sem).wait()

        def _scatter(j, buf):
            pltpu.sync_copy(buf, agg_sh.at[dst_v.at[j]], add=True)

        _gather(0, rows0, sem0)
        _gather(1, rows1, sem1)

        def _chunk3(t, _):
            j0 = 3 * t
            _gwait(rows0, sem0)
            _gather(j0 + 2, rows2, sem2)
            _scatter(j0, rows0)
            _gwait(rows1, sem1)
            _gather(jnp.minimum(j0 + 3, BASE_CH - 1), rows0, sem0)
            _scatter(j0 + 1, rows1)
            _gwait(rows2, sem2)
            _gather(jnp.minimum(j0 + 4, BASE_CH - 1), rows1, sem1)
            _scatter(j0 + 2, rows2)
            return 0
        lax.fori_loop(0, BASE_CH // 3, _chunk3, 0)
        _gwait(rows0, sem0)
        _gwait(rows1, sem1)

        @pl.when(has_extra)
        def _tail():
            pltpu.async_copy(h_sh.at[src_v.at[BASE_CH]], rows0,
                             sem0).wait()
            _scatter(BASE_CH, rows0)

        plsc.subcore_barrier()

        pltpu.sync_copy(agg_sh.at[pl.ds(base, RPT)],
                        aggp_hbm.at[cid, pl.ds(base, RPT)])
        if echo_h:
            @pl.when(cid == 0)
            def _echo():
                pltpu.sync_copy(h_sh.at[pl.ds(base, RPT)],
                                echo_hbm.at[pl.ds(base, RPT)])

    return pl.kernel(body, out_type=out_type, mesh=_MESH,
                     scratch_types=scratch, compiler_params=_SC_PARAMS)


_seg_echo = _make_seg(True)
_seg_plain = _make_seg(False)


def _proj_body(x_ref, w_ref, b_ref, o_ref):
    o_ref[...] = jnp.maximum(
        jnp.dot(x_ref[...], w_ref[...], preferred_element_type=jnp.float32)
        + b_ref[...], 0.0)


def _inv_body(degp_ref, o_ref):
    o_ref[...] = jnp.maximum(degp_ref[0] + degp_ref[1], 1.0)


def _pair_mean(aggp_ref, inv_ref):
    a = aggp_ref[0] + aggp_ref[1]
    iv = inv_ref[...]
    degb = jnp.concatenate(
        [jnp.broadcast_to(iv[:, 0:1], (a.shape[0], H)),
         jnp.broadcast_to(iv[:, DEGW:DEGW + 1], (a.shape[0], H))], axis=1)
    return a / degb


def _upd_body(aggp_ref, inv_ref, h_ref, wl_ref, bl_ref, wr_ref, o_ref):
    mean2 = _pair_mean(aggp_ref, inv_ref)
    o_ref[...] = jnp.maximum(
        jnp.dot(mean2, wl_ref[...], preferred_element_type=jnp.float32)
        + jnp.dot(h_ref[...], wr_ref[...], preferred_element_type=jnp.float32)
        + bl_ref[...], 0.0)


def _head_body(aggp_ref, inv_ref, h_ref, wl_ref, bl_ref, wr_ref,
               wc_ref, bc_ref, o_ref):
    mean2 = _pair_mean(aggp_ref, inv_ref)
    h2 = jnp.maximum(
        jnp.dot(mean2, wl_ref[...], preferred_element_type=jnp.float32)
        + jnp.dot(h_ref[...], wr_ref[...], preferred_element_type=jnp.float32)
        + bl_ref[...], 0.0)
    s = h2 * wc_ref[...]
    o_ref[...] = jnp.concatenate(
        [jnp.sum(s[:, :H], axis=1, keepdims=True),
         jnp.sum(s[:, H:], axis=1, keepdims=True)], axis=1) + bc_ref[0, 0]


def _full(shape):
    return pl.BlockSpec(shape, lambda i: tuple(0 for _ in shape))


def _blockdiag(w):
    z = jnp.zeros((H, H), jnp.float32)
    return jnp.concatenate(
        [jnp.concatenate([w, z], axis=1),
         jnp.concatenate([z, w], axis=1)], axis=0)


def kernel(x, edge_index, W_proj, b_proj, Wl1, bl1, Wr1, Wl2, bl2, Wr2,
           Wc, bc):
    e3 = edge_index.reshape(2, NCHUNK, CB)
    z64 = jnp.zeros((RPT, H), jnp.float32)
    z8 = jnp.zeros((RPTD, DEGW), jnp.float32)
    ones = jnp.ones((CB, DEGW), jnp.float32)

    degp = _deg_seg(e3, z8, ones)

    grid = (N // BR,)
    h = pl.pallas_call(
        _proj_body,
        grid=grid,
        in_specs=[pl.BlockSpec((BR, D_IN), lambda i: (i, 0)),
                  _full((D_IN, H)), _full((1, H))],
        out_specs=pl.BlockSpec((BR, H), lambda i: (i, 0)),
        out_shape=jax.ShapeDtypeStruct((N, H), jnp.float32),
    )(x, W_proj, b_proj.reshape(1, H))

    inv128 = pl.pallas_call(
        _inv_body,
        grid=(1,),
        in_specs=[_full((NC, ND * DEGW // H2, H2))],
        out_specs=pl.BlockSpec((ND * DEGW // H2, H2), lambda i: (0, 0)),
        out_shape=jax.ShapeDtypeStruct((ND * DEGW // H2, H2), jnp.float32),
    )(degp.reshape(NC, ND * DEGW // H2, H2))
    inv16 = inv128.reshape(ND // 2, 2 * DEGW)

    aggp1, h_echo = _seg_echo(h, e3, z64)

    aggp2_spec = pl.BlockSpec((NC, BR2, H2), lambda i: (0, i, 0))
    inv_spec = pl.BlockSpec((BR2, 2 * DEGW), lambda i: (i, 0))
    row2_spec = pl.BlockSpec((BR2, H2), lambda i: (i, 0))

    h1_2d = pl.pallas_call(
        _upd_body,
        grid=grid,
        in_specs=[aggp2_spec, inv_spec, row2_spec,
                  _full((H2, H2)), _full((1, H2)), _full((H2, H2))],
        out_specs=row2_spec,
        out_shape=jax.ShapeDtypeStruct((N2, H2), jnp.float32),
    )(aggp1.reshape(NC, N2, H2), inv16, h_echo.reshape(N2, H2),
      _blockdiag(Wl1), jnp.concatenate([bl1, bl1]).reshape(1, H2),
      _blockdiag(Wr1))

    aggp2 = _seg_plain(h1_2d.reshape(N, H), e3, z64)

    out2 = pl.pallas_call(
        _head_body,
        grid=grid,
        in_specs=[aggp2_spec, inv_spec, row2_spec,
                  _full((H2, H2)), _full((1, H2)), _full((H2, H2)),
                  _full((1, H2)), _full((1, 1))],
        out_specs=pl.BlockSpec((BR2, 2), lambda i: (i, 0)),
        out_shape=jax.ShapeDtypeStruct((N2, 2), jnp.float32),
    )(aggp2.reshape(NC, N2, H2), inv16, h1_2d,
      _blockdiag(Wl2), jnp.concatenate([bl2, bl2]).reshape(1, H2),
      _blockdiag(Wr2), jnp.concatenate([Wc[:, 0], Wc[:, 0]]).reshape(1, H2),
      bc.reshape(1, 1))

    return out2.reshape(N)


# speedup vs baseline: 1.0585x; 1.0249x over previous
import jax
import jax.numpy as jnp
from jax import lax
from jax.experimental import pallas as pl
from jax.experimental.pallas import tpu as pltpu
from jax.experimental.pallas import tpu_sc as plsc

N = 10000
E = 320000
D_IN = 128
H = 64
H2 = 2 * H
N2 = N // 2

NC = 2
NS = 16
NW = NC * NS
CB = 128
NCHUNK = E // CB
BASE_CH = NCHUNK // NW
EXTRA_W = NCHUNK - BASE_CH * NW
RPT = N // NS
DEGW = 8
ND = 10240
RPTD = ND // NS
BR = 2000
BR2 = BR // 2

_MESH = plsc.VectorSubcoreMesh(
    core_axis_name="c", subcore_axis_name="s",
    num_cores=NC, num_subcores=NS)
_SC_PARAMS = pltpu.CompilerParams(use_tc_tiling_on_sc=False)


def _worker_chunks(wid):
    start = BASE_CH * wid + jnp.maximum(wid - (NW - EXTRA_W), 0)
    has_extra = wid >= NW - EXTRA_W
    return start, has_extra


def _deg_body(e_hbm, z8_hbm, ones_hbm, degp_hbm, dst_v, ones_v, deg_sh,
              gsem):
    cid = lax.axis_index("c")
    sid = lax.axis_index("s")
    wid = sid * NC + cid
    based = sid * RPTD
    start, has_extra = _worker_chunks(wid)

    pltpu.sync_copy(z8_hbm, deg_sh.at[pl.ds(based, RPTD)])
    pltpu.sync_copy(ones_hbm, ones_v)
    pltpu.sync_copy(e_hbm.at[1, pl.ds(start, BASE_CH + 1)], dst_v)
    plsc.subcore_barrier()

    def _scat(j):
        pltpu.async_copy(ones_v, deg_sh.at[dst_v.at[j]], gsem, add=True)

    def _swait():
        pltpu.make_async_copy(ones_v, deg_sh.at[dst_v.at[0]], gsem).wait()

    _scat(0)

    def _step(j, _):
        _scat(j + 1)
        _swait()
        return 0
    lax.fori_loop(0, BASE_CH - 1, _step, 0)
    _swait()

    @pl.when(has_extra)
    def _tail():
        pltpu.async_copy(ones_v, deg_sh.at[dst_v.at[BASE_CH]], gsem,
                         add=True).wait()

    plsc.subcore_barrier()
    pltpu.sync_copy(deg_sh.at[pl.ds(based, RPTD)],
                    degp_hbm.at[cid, pl.ds(based, RPTD)])


_deg_seg = pl.kernel(
    _deg_body,
    out_type=jax.ShapeDtypeStruct((NC, ND, DEGW), jnp.float32),
    mesh=_MESH,
    scratch_types=[
        pltpu.VMEM((BASE_CH + 1, CB), jnp.int32),
        pltpu.VMEM((CB, DEGW), jnp.float32),
        pltpu.VMEM_SHARED((ND, DEGW), jnp.float32),
        pltpu.SemaphoreType.DMA,
    ],
    compiler_params=_SC_PARAMS)


def _make_seg(echo_h):
    agg_ty = jax.ShapeDtypeStruct((NC, N, H), jnp.float32)
    if echo_h:
        out_type = (agg_ty, jax.ShapeDtypeStruct((N, H), jnp.float32))
    else:
        out_type = agg_ty
    scratch = [
        pltpu.VMEM((BASE_CH + 1, CB), jnp.int32),
        pltpu.VMEM((BASE_CH + 1, CB), jnp.int32),
        pltpu.VMEM((CB, H), jnp.float32),
        pltpu.VMEM((CB, H), jnp.float32),
        pltpu.VMEM((CB, H), jnp.float32),
        pltpu.VMEM_SHARED((N, H), jnp.float32),
        pltpu.VMEM_SHARED((N, H), jnp.float32),
        pltpu.SemaphoreType.DMA,
        pltpu.SemaphoreType.DMA,
        pltpu.SemaphoreType.DMA,
    ]

    def body(*refs):
        if echo_h:
            (h_hbm, e_hbm, z64_hbm, aggp_hbm, echo_hbm,
             src_v, dst_v, rows0, rows1, rows2, agg_sh, h_sh,
             sem0, sem1, sem2) = refs
        else:
            (h_hbm, e_hbm, z64_hbm, aggp_hbm,
             src_v, dst_v, rows0, rows1, rows2, agg_sh, h_sh,
             sem0, sem1, sem2) = refs
            echo_hbm = None

        cid = lax.axis_index("c")
        sid = lax.axis_index("s")
        wid = sid * NC + cid
        base = sid * RPT
        start, has_extra = _worker_chunks(wid)

        pltpu.sync_copy(z64_hbm, agg_sh.at[pl.ds(base, RPT)])
        pltpu.sync_copy(e_hbm.at[0, pl.ds(start, BASE_CH + 1)], src_v)
        pltpu.sync_copy(e_hbm.at[1, pl.ds(start, BASE_CH + 1)], dst_v)
        pltpu.sync_copy(h_hbm.at[pl.ds(base, RPT)], h_sh.at[pl.ds(base, RPT)])

        plsc.subcore_barrier()

        def _gather(j, buf, sem):
            pltpu.async_copy(h_sh.at[src_v.at[j]], buf, sem)

        def _gwait(buf, sem):
            pltpu.make_async_copy(h_sh.at[src_v.at[0]], buf, sem).wait()

        def _scatter(j, buf):
            pltpu.sync_copy(buf, agg_sh.at[dst_v.at[j]], add=True)

        _gather(0, rows0, sem0)
        _gather(1, rows1, sem1)

        def _chunk3(t, _):
            j0 = 3 * t
            _gwait(rows0, sem0)
            _gather(j0 + 2, rows2, sem2)
            _scatter(j0, rows0)
            _gwait(rows1, sem1)
            _gather(jnp.minimum(j0 + 3, BASE_CH - 1), rows0, sem0)
            _scatter(j0 + 1, rows1)
            _gwait(rows2, sem2)
            _gather(jnp.minimum(j0 + 4, BASE_CH - 1), rows1, sem1)
            _scatter(j0 + 2, rows2)
            return 0
        lax.fori_loop(0, BASE_CH // 3, _chunk3, 0)
        _gwait(rows0, sem0)
        _gwait(rows1, sem1)

        @pl.when(has_extra)
        def _tail():
            pltpu.async_copy(h_sh.at[src_v.at[BASE_CH]], rows0,
                             sem0).wait()
            _scatter(BASE_CH, rows0)

        plsc.subcore_barrier()

        pltpu.sync_copy(agg_sh.at[pl.ds(base, RPT)],
                        aggp_hbm.at[cid, pl.ds(base, RPT)])
        if echo_h:
            @pl.when(cid == 0)
            def _echo():
                pltpu.sync_copy(h_sh.at[pl.ds(base, RPT)],
                                echo_hbm.at[pl.ds(base, RPT)])

    return pl.kernel(body, out_type=out_type, mesh=_MESH,
                     scratch_types=scratch, compiler_params=_SC_PARAMS)


_seg_echo = _make_seg(True)
_seg_plain = _make_seg(False)


def _proj_body(x_ref, w_ref, b_ref, o_ref):
    o_ref[...] = jnp.maximum(
        jnp.dot(x_ref[...], w_ref[...], preferred_element_type=jnp.float32)
        + b_ref[...], 0.0)


def _inv_body(degp_ref, o_ref):
    o_ref[...] = jnp.maximum(degp_ref[0] + degp_ref[1], 1.0)


def _pair_mean(aggp_ref, inv_ref):
    a = aggp_ref[0] + aggp_ref[1]
    iv = inv_ref[...]
    degb = jnp.concatenate(
        [jnp.broadcast_to(iv[:, 0:1], (a.shape[0], H)),
         jnp.broadcast_to(iv[:, DEGW:DEGW + 1], (a.shape[0], H))], axis=1)
    return a / degb


def _upd_body(aggp_ref, inv_ref, h_ref, wl_ref, bl_ref, wr_ref, o_ref):
    mean2 = _pair_mean(aggp_ref, inv_ref)
    o_ref[...] = jnp.maximum(
        jnp.dot(mean2, wl_ref[...], preferred_element_type=jnp.float32)
        + jnp.dot(h_ref[...], wr_ref[...], preferred_element_type=jnp.float32)
        + bl_ref[...], 0.0)


def _head_body(aggp_ref, inv_ref, h_ref, wl_ref, bl_ref, wr_ref,
               wc_ref, bc_ref, o_ref):
    mean2 = _pair_mean(aggp_ref, inv_ref)
    h2 = jnp.maximum(
        jnp.dot(mean2, wl_ref[...], preferred_element_type=jnp.float32)
        + jnp.dot(h_ref[...], wr_ref[...], preferred_element_type=jnp.float32)
        + bl_ref[...], 0.0)
    o_ref[...] = (
        jnp.dot(h2, wc_ref[...], preferred_element_type=jnp.float32)
        + bc_ref[0, 0])


def _full(shape):
    return pl.BlockSpec(shape, lambda i: tuple(0 for _ in shape))


def _blockdiag(w):
    z = jnp.zeros((H, H), jnp.float32)
    return jnp.concatenate(
        [jnp.concatenate([w, z], axis=1),
         jnp.concatenate([z, w], axis=1)], axis=0)


def kernel(x, edge_index, W_proj, b_proj, Wl1, bl1, Wr1, Wl2, bl2, Wr2,
           Wc, bc):
    e3 = edge_index.reshape(2, NCHUNK, CB)
    z64 = jnp.zeros((RPT, H), jnp.float32)
    z8 = jnp.zeros((RPTD, DEGW), jnp.float32)
    ones = jnp.ones((CB, DEGW), jnp.float32)

    degp = _deg_seg(e3, z8, ones)

    grid = (N // BR,)
    h = pl.pallas_call(
        _proj_body,
        grid=grid,
        in_specs=[pl.BlockSpec((BR, D_IN), lambda i: (i, 0)),
                  _full((D_IN, H)), _full((1, H))],
        out_specs=pl.BlockSpec((BR, H), lambda i: (i, 0)),
        out_shape=jax.ShapeDtypeStruct((N, H), jnp.float32),
    )(x, W_proj, b_proj.reshape(1, H))

    inv128 = pl.pallas_call(
        _inv_body,
        grid=(1,),
        in_specs=[_full((NC, ND * DEGW // H2, H2))],
        out_specs=pl.BlockSpec((ND * DEGW // H2, H2), lambda i: (0, 0)),
        out_shape=jax.ShapeDtypeStruct((ND * DEGW // H2, H2), jnp.float32),
    )(degp.reshape(NC, ND * DEGW // H2, H2))
    inv16 = inv128.reshape(ND // 2, 2 * DEGW)

    aggp1, h_echo = _seg_echo(h, e3, z64)

    aggp2_spec = pl.BlockSpec((NC, BR2, H2), lambda i: (0, i, 0))
    inv_spec = pl.BlockSpec((BR2, 2 * DEGW), lambda i: (i, 0))
    row2_spec = pl.BlockSpec((BR2, H2), lambda i: (i, 0))

    h1_2d = pl.pallas_call(
        _upd_body,
        grid=grid,
        in_specs=[aggp2_spec, inv_spec, row2_spec,
                  _full((H2, H2)), _full((1, H2)), _full((H2, H2))],
        out_specs=row2_spec,
        out_shape=jax.ShapeDtypeStruct((N2, H2), jnp.float32),
    )(aggp1.reshape(NC, N2, H2), inv16, h_echo.reshape(N2, H2),
      _blockdiag(Wl1), jnp.concatenate([bl1, bl1]).reshape(1, H2),
      _blockdiag(Wr1))

    aggp2 = _seg_plain(h1_2d.reshape(N, H), e3, z64)

    out2 = pl.pallas_call(
        _head_body,
        grid=grid,
        in_specs=[aggp2_spec, inv_spec, row2_spec,
                  _full((H2, H2)), _full((1, H2)), _full((H2, H2)),
                  _full((H2, 2)), _full((1, 1))],
        out_specs=pl.BlockSpec((BR2, 2), lambda i: (i, 0)),
        out_shape=jax.ShapeDtypeStruct((N2, 2), jnp.float32),
    )(aggp2.reshape(NC, N2, H2), inv16, h1_2d,
      _blockdiag(Wl2), jnp.concatenate([bl2, bl2]).reshape(1, H2),
      _blockdiag(Wr2),
      jnp.concatenate(
          [jnp.concatenate([Wc, jnp.zeros((H, 1))], axis=1),
           jnp.concatenate([jnp.zeros((H, 1)), Wc], axis=1)], axis=0),
      bc.reshape(1, 1))

    return out2.reshape(N)
